# baseline (speedup 1.0000x reference)
import jax
import jax.numpy as jnp
from jax.experimental import pallas as pl
from jax.experimental.pallas import tpu as pltpu

N_INST = 8192
N_NET = 4096
EMB = 64
NUM_VN = 16

BR = 1024
KK = 2048
BN = 512
BC = 1024


def _lrelu(v):
    return jnp.where(v >= 0, v, 0.1 * v)


def _dot(a, b):
    return jnp.dot(a, b, preferred_element_type=jnp.float32)


def _enc(x, w1, b1, w2, b2):
    h = _lrelu(_dot(x, w1.T) + b1)
    return _lrelu(_dot(h, w2.T) + b2)


def _phi(hn, p1, pb1, p2, pb2):
    return _dot(jax.nn.relu(_dot(hn, p1.T) + pb1), p2.T) + pb2


def _tail_math(h_in, hd, hs0, psi1, psib1, psi2, psib2,
               m1, mb1, m2, mb2, g, be):
    hs = _dot(jax.nn.relu(_dot(hs0, psi1.T) + psib1), psi2.T) + psib2
    hc = jnp.concatenate([h_in, hd, hs], axis=1)
    hm = jax.nn.relu(_dot(hc, m1.T) + mb1)
    ho = _dot(hm, m2.T) + mb2
    mu = jnp.mean(ho, axis=-1, keepdims=True)
    var = jnp.mean((ho - mu) ** 2, axis=-1, keepdims=True)
    ln = (ho - mu) / jnp.sqrt(var + 1e-5) * g + be
    return ho, _lrelu(ln) + h_in


def _layer0_body(x_ref, xn_ref, drive_ref, sink_ref, oh_ref, vnt_ref,
                 e1_ref, eb1_ref, e2_ref, eb2_ref,
                 n1_ref, nb1_ref, n2_ref, nb2_ref,
                 p1_ref, pb1_ref, p2_ref, pb2_ref,
                 psi1_ref, psib1_ref, psi2_ref, psib2_ref,
                 m1_ref, mb1_ref, m2_ref, mb2_ref, g_ref, be_ref,
                 q1_ref, qb1_ref, q2_ref, qb2_ref,
                 ho_ref, hin_ref, hpre_ref, di8_ref, si8_ref,
                 pool_ref, vn_next_ref, hd_acc, hs_acc):
    i = pl.program_id(0)
    k = pl.program_id(1)
    hn0 = _enc(xn_ref[...], n1_ref[...], nb1_ref[...], n2_ref[...],
               nb2_ref[...])
    na0 = _phi(hn0, p1_ref[...], pb1_ref[...], p2_ref[...], pb2_ref[...])
    pd = _dot(drive_ref[...], na0)
    ps = _dot(sink_ref[...], na0)
    di8_ref[...] = drive_ref[...].astype(jnp.int8)
    si8_ref[...] = sink_ref[...].astype(jnp.int8)

    @pl.when(k == 0)
    def _():
        hd_acc[...] = pd
        hs_acc[...] = ps

    @pl.when(k > 0)
    def _():
        hd_acc[...] += pd
        hs_acc[...] += ps

    @pl.when(k == pl.num_programs(1) - 1)
    def _():
        h0 = _enc(x_ref[...], e1_ref[...], eb1_ref[...], e2_ref[...],
                  eb2_ref[...])
        h_in = h0 + _dot(oh_ref[...], vnt_ref[...])
        hin_ref[...] = h_in
        hpre, hout = _tail_math(
            h_in, hd_acc[...], hs_acc[...], psi1_ref[...], psib1_ref[...],
            psi2_ref[...], psib2_ref[...], m1_ref[...], mb1_ref[...],
            m2_ref[...], mb2_ref[...], g_ref[...], be_ref[...])
        hpre_ref[...] = hpre
        ho_ref[...] = hout

        ones = jnp.ones((h_in.shape[0], EMB), jnp.float32)
        hp = jnp.concatenate([h_in, ones], axis=1)
        contrib = _dot(oh_ref[...].T, hp)

        @pl.when(i == 0)
        def _():
            pool_ref[...] = jnp.zeros_like(pool_ref)

        pool_ref[...] += contrib

        @pl.when(i == pl.num_programs(0) - 1)
        def _():
            pool = pool_ref[...]
            counts = jnp.maximum(pool[:, EMB:EMB + 1], 1.0)
            vn_in = pool[:, :EMB] / counts + vnt_ref[...]
            t = _lrelu(_dot(vn_in, q1_ref[...].T) + qb1_ref[...])
            vn_next_ref[...] = _lrelu(_dot(t, q2_ref[...].T) + qb2_ref[...])


def _net_body(adj_ref, hpre_ref, xn_ref,
              n1_ref, nb1_ref, n2_ref, nb2_ref,
              p1_ref, pb1_ref, p2_ref, pb2_ref, na1_ref):
    hn0 = _enc(xn_ref[...], n1_ref[...], nb1_ref[...], n2_ref[...],
               nb2_ref[...])
    hn1 = _dot(adj_ref[...], hpre_ref[...]) + hn0
    na1_ref[...] = _phi(hn1, p1_ref[...], pb1_ref[...], p2_ref[...],
                        pb2_ref[...])


def _layer1_body(drive_ref, sink_ref, na_ref, hb_ref, oh_ref, vnt_ref,
                 psi1_ref, psib1_ref, psi2_ref, psib2_ref,
                 m1_ref, mb1_ref, m2_ref, mb2_ref, g_ref, be_ref,
                 ho_ref, hin_ref):
    h_in = hb_ref[...] + _dot(oh_ref[...], vnt_ref[...])
    hin_ref[...] = h_in
    na_bf = na_ref[...].astype(jnp.bfloat16)
    hd = _dot(drive_ref[...].astype(jnp.bfloat16), na_bf)
    hs0 = _dot(sink_ref[...].astype(jnp.bfloat16), na_bf)
    _, hout = _tail_math(
        h_in, hd, hs0, psi1_ref[...], psib1_ref[...], psi2_ref[...],
        psib2_ref[...], m1_ref[...], mb1_ref[...], m2_ref[...], mb2_ref[...],
        g_ref[...], be_ref[...])
    ho_ref[...] = hout


def _full(shape):
    return pl.BlockSpec(shape, lambda i: tuple(0 for _ in shape))


def _rows(bs, width):
    return pl.BlockSpec((bs, width), lambda i: (i, 0))


def kernel(x, x_net, net_inst_adj, inst_net_adj_v_drive, inst_net_adj_v_sink,
           batch, num_vn, params):
    p = params
    r2 = lambda a: a.reshape(1, -1)
    oh = (batch[:, None] == jnp.arange(NUM_VN, dtype=batch.dtype)[None, :]
          ).astype(jnp.float32)
    vn0 = jnp.tile(p["vn_emb"], (NUM_VN, 1)) + 0.0 * num_vn
    L0, L1 = p["layers"][0], p["layers"][1]
    q0 = p["vn_mlp"][0]

    rows0 = lambda bs, w: pl.BlockSpec((bs, w), lambda i, k: (i, 0))
    full0 = lambda shape: pl.BlockSpec(shape,
                                       lambda i, k: tuple(0 for _ in shape))
    h_out0, h_in0, h_pre0, drive_i8, sink_i8, _, vn1 = pl.pallas_call(
        _layer0_body,
        grid=(N_INST // BR, N_NET // KK),
        scratch_shapes=[pltpu.VMEM((BR, EMB), jnp.float32),
                        pltpu.VMEM((BR, EMB), jnp.float32)],
        in_specs=[rows0(BR, x.shape[1]),
                  pl.BlockSpec((KK, x_net.shape[1]), lambda i, k: (k, 0)),
                  pl.BlockSpec((BR, KK), lambda i, k: (i, k)),
                  pl.BlockSpec((BR, KK), lambda i, k: (i, k)),
                  rows0(BR, NUM_VN), full0((NUM_VN, EMB)),
                  full0(p["enc_W1"].shape), full0((1, 2 * EMB)),
                  full0(p["enc_W2"].shape), full0((1, EMB)),
                  full0(p["encnet_W1"].shape), full0((1, EMB)),
                  full0(p["encnet_W2"].shape), full0((1, EMB)),
                  full0(L0["phi_W1"].shape), full0((1, EMB)),
                  full0(L0["phi_W2"].shape), full0((1, EMB)),
                  full0(L0["psi_W1"].shape), full0((1, EMB)),
                  full0(L0["psi_W2"].shape), full0((1, EMB)),
                  full0(L0["mlp_W1"].shape), full0((1, 3 * EMB)),
                  full0(L0["mlp_W2"].shape), full0((1, EMB)),
                  full0((1, EMB)), full0((1, EMB)),
                  full0(q0["W1"].shape), full0((1, 2 * EMB)),
                  full0(q0["W2"].shape), full0((1, EMB))],
        out_specs=[rows0(BR, EMB), rows0(BR, EMB), rows0(BR, EMB),
                   pl.BlockSpec((BR, KK), lambda i, k: (i, k)),
                   pl.BlockSpec((BR, KK), lambda i, k: (i, k)),
                   full0((NUM_VN, 2 * EMB)), full0((NUM_VN, EMB))],
        out_shape=[jax.ShapeDtypeStruct((N_INST, EMB), jnp.float32),
                   jax.ShapeDtypeStruct((N_INST, EMB), jnp.float32),
                   jax.ShapeDtypeStruct((N_INST, EMB), jnp.float32),
                   jax.ShapeDtypeStruct((N_INST, N_NET), jnp.int8),
                   jax.ShapeDtypeStruct((N_INST, N_NET), jnp.int8),
                   jax.ShapeDtypeStruct((NUM_VN, 2 * EMB), jnp.float32),
                   jax.ShapeDtypeStruct((NUM_VN, EMB), jnp.float32)],
    )(x, x_net, inst_net_adj_v_drive, inst_net_adj_v_sink, oh, vn0,
      p["enc_W1"], r2(p["enc_b1"]), p["enc_W2"], r2(p["enc_b2"]),
      p["encnet_W1"], r2(p["encnet_b1"]), p["encnet_W2"], r2(p["encnet_b2"]),
      L0["phi_W1"], r2(L0["phi_b1"]), L0["phi_W2"], r2(L0["phi_b2"]),
      L0["psi_W1"], r2(L0["psi_b1"]), L0["psi_W2"], r2(L0["psi_b2"]),
      L0["mlp_W1"], r2(L0["mlp_b1"]), L0["mlp_W2"], r2(L0["mlp_b2"]),
      r2(L0["ln_g"]), r2(L0["ln_b"]),
      q0["W1"], r2(q0["b1"]), q0["W2"], r2(q0["b2"]))

    na1 = pl.pallas_call(
        _net_body,
        grid=(N_NET // BN,),
        in_specs=[_rows(BN, N_INST), _full((N_INST, EMB)),
                  _rows(BN, x_net.shape[1]),
                  _full(p["encnet_W1"].shape), _full((1, EMB)),
                  _full(p["encnet_W2"].shape), _full((1, EMB)),
                  _full(L1["phi_W1"].shape), _full((1, EMB)),
                  _full(L1["phi_W2"].shape), _full((1, EMB))],
        out_specs=_rows(BN, EMB),
        out_shape=jax.ShapeDtypeStruct((N_NET, EMB), jnp.float32),
    )(net_inst_adj, h_pre0, x_net,
      p["encnet_W1"], r2(p["encnet_b1"]), p["encnet_W2"], r2(p["encnet_b2"]),
      L1["phi_W1"], r2(L1["phi_b1"]), L1["phi_W2"], r2(L1["phi_b2"]))

    h_out1, h_in1 = pl.pallas_call(
        _layer1_body,
        grid=(N_INST // BC,),
        in_specs=[_rows(BC, N_NET), _rows(BC, N_NET), _full((N_NET, EMB)),
                  _rows(BC, EMB), _rows(BC, NUM_VN), _full((NUM_VN, EMB)),
                  _full(L1["psi_W1"].shape), _full((1, EMB)),
                  _full(L1["psi_W2"].shape), _full((1, EMB)),
                  _full(L1["mlp_W1"].shape), _full((1, 3 * EMB)),
                  _full(L1["mlp_W2"].shape), _full((1, EMB)),
                  _full((1, EMB)), _full((1, EMB))],
        out_specs=[_rows(BC, EMB), _rows(BC, EMB)],
        out_shape=[jax.ShapeDtypeStruct((N_INST, EMB), jnp.float32),
                   jax.ShapeDtypeStruct((N_INST, EMB), jnp.float32)],
    )(drive_i8, sink_i8, na1, h_out0, oh, vn1,
      L1["psi_W1"], r2(L1["psi_b1"]), L1["psi_W2"], r2(L1["psi_b2"]),
      L1["mlp_W1"], r2(L1["mlp_b1"]), L1["mlp_W2"], r2(L1["mlp_b2"]),
      r2(L1["ln_g"]), r2(L1["ln_b"]))

    return jnp.concatenate([h_in0, h_in1, h_out1], axis=1)


# speedup vs baseline: 1.0041x; 1.0041x over previous
import jax
import jax.numpy as jnp
from jax.experimental import pallas as pl

N_INST = 8192
N_NET = 4096
EMB = 64
NUM_VN = 16

BR = 512
BN = 512
BC = 1024


def _lrelu(v):
    return jnp.where(v >= 0, v, 0.1 * v)


def _dot(a, b):
    return jnp.dot(a, b, preferred_element_type=jnp.float32)


def _enc(x, w1, b1, w2, b2):
    h = _lrelu(_dot(x, w1.T) + b1)
    return _lrelu(_dot(h, w2.T) + b2)


def _phi(hn, p1, pb1, p2, pb2):
    return _dot(jax.nn.relu(_dot(hn, p1.T) + pb1), p2.T) + pb2


def _tail_math(h_in, hd, hs0, psi1, psib1, psi2, psib2,
               m1, mb1, m2, mb2, g, be):
    hs = _dot(jax.nn.relu(_dot(hs0, psi1.T) + psib1), psi2.T) + psib2
    hc = jnp.concatenate([h_in, hd, hs], axis=1)
    hm = jax.nn.relu(_dot(hc, m1.T) + mb1)
    ho = _dot(hm, m2.T) + mb2
    mu = jnp.mean(ho, axis=-1, keepdims=True)
    var = jnp.mean((ho - mu) ** 2, axis=-1, keepdims=True)
    ln = (ho - mu) / jnp.sqrt(var + 1e-5) * g + be
    return ho, _lrelu(ln) + h_in


def _layer0_body(x_ref, xn_ref, drive_ref, sink_ref, oh_ref, vnt_ref,
                 e1_ref, eb1_ref, e2_ref, eb2_ref,
                 n1_ref, nb1_ref, n2_ref, nb2_ref,
                 p1_ref, pb1_ref, p2_ref, pb2_ref,
                 psi1_ref, psib1_ref, psi2_ref, psib2_ref,
                 m1_ref, mb1_ref, m2_ref, mb2_ref, g_ref, be_ref,
                 q1_ref, qb1_ref, q2_ref, qb2_ref,
                 ho_ref, hin_ref, hpre_ref, di8_ref, si8_ref,
                 pool_ref, vn_next_ref):
    i = pl.program_id(0)
    h0 = _enc(x_ref[...], e1_ref[...], eb1_ref[...], e2_ref[...], eb2_ref[...])
    hn0 = _enc(xn_ref[...], n1_ref[...], nb1_ref[...], n2_ref[...],
               nb2_ref[...])
    na0 = _phi(hn0, p1_ref[...], pb1_ref[...], p2_ref[...], pb2_ref[...])
    h_in = h0 + _dot(oh_ref[...], vnt_ref[...])
    hin_ref[...] = h_in
    hd = _dot(drive_ref[...], na0)
    hs0 = _dot(sink_ref[...], na0)
    hpre, hout = _tail_math(
        h_in, hd, hs0, psi1_ref[...], psib1_ref[...], psi2_ref[...],
        psib2_ref[...], m1_ref[...], mb1_ref[...], m2_ref[...], mb2_ref[...],
        g_ref[...], be_ref[...])
    hpre_ref[...] = hpre
    ho_ref[...] = hout
    di8_ref[...] = drive_ref[...].astype(jnp.int8)
    si8_ref[...] = sink_ref[...].astype(jnp.int8)

    ones = jnp.ones((h_in.shape[0], EMB), jnp.float32)
    hp = jnp.concatenate([h_in, ones], axis=1)
    contrib = _dot(oh_ref[...].T, hp)

    @pl.when(i == 0)
    def _():
        pool_ref[...] = jnp.zeros_like(pool_ref)

    pool_ref[...] += contrib

    @pl.when(i == pl.num_programs(0) - 1)
    def _():
        pool = pool_ref[...]
        counts = jnp.maximum(pool[:, EMB:EMB + 1], 1.0)
        vn_in = pool[:, :EMB] / counts + vnt_ref[...]
        t = _lrelu(_dot(vn_in, q1_ref[...].T) + qb1_ref[...])
        vn_next_ref[...] = _lrelu(_dot(t, q2_ref[...].T) + qb2_ref[...])


def _net_body(adj_ref, hpre_ref, xn_ref,
              n1_ref, nb1_ref, n2_ref, nb2_ref,
              p1_ref, pb1_ref, p2_ref, pb2_ref, na1_ref):
    hn0 = _enc(xn_ref[...], n1_ref[...], nb1_ref[...], n2_ref[...],
               nb2_ref[...])
    hn1 = _dot(adj_ref[...], hpre_ref[...]) + hn0
    na1_ref[...] = _phi(hn1, p1_ref[...], pb1_ref[...], p2_ref[...],
                        pb2_ref[...])


def _layer1_body(drive_ref, sink_ref, na_ref, hb_ref, oh_ref, vnt_ref,
                 psi1_ref, psib1_ref, psi2_ref, psib2_ref,
                 m1_ref, mb1_ref, m2_ref, mb2_ref, g_ref, be_ref,
                 ho_ref, hin_ref):
    h_in = hb_ref[...] + _dot(oh_ref[...], vnt_ref[...])
    hin_ref[...] = h_in
    na_bf = na_ref[...].astype(jnp.bfloat16)
    hd = _dot(drive_ref[...].astype(jnp.bfloat16), na_bf)
    hs0 = _dot(sink_ref[...].astype(jnp.bfloat16), na_bf)
    _, hout = _tail_math(
        h_in, hd, hs0, psi1_ref[...], psib1_ref[...], psi2_ref[...],
        psib2_ref[...], m1_ref[...], mb1_ref[...], m2_ref[...], mb2_ref[...],
        g_ref[...], be_ref[...])
    ho_ref[...] = hout


def _full(shape):
    return pl.BlockSpec(shape, lambda i: tuple(0 for _ in shape))


def _rows(bs, width):
    return pl.BlockSpec((bs, width), lambda i: (i, 0))


def kernel(x, x_net, net_inst_adj, inst_net_adj_v_drive, inst_net_adj_v_sink,
           batch, num_vn, params):
    p = params
    r2 = lambda a: a.reshape(1, -1)
    oh = (batch[:, None] == jnp.arange(NUM_VN, dtype=batch.dtype)[None, :]
          ).astype(jnp.float32)
    vn0 = jnp.tile(p["vn_emb"], (NUM_VN, 1)) + 0.0 * num_vn
    L0, L1 = p["layers"][0], p["layers"][1]
    q0 = p["vn_mlp"][0]

    h_out0, h_in0, h_pre0, drive_i8, sink_i8, _, vn1 = pl.pallas_call(
        _layer0_body,
        grid=(N_INST // BR,),
        in_specs=[_rows(BR, x.shape[1]), _full(x_net.shape),
                  _rows(BR, N_NET), _rows(BR, N_NET),
                  _rows(BR, NUM_VN), _full((NUM_VN, EMB)),
                  _full(p["enc_W1"].shape), _full((1, 2 * EMB)),
                  _full(p["enc_W2"].shape), _full((1, EMB)),
                  _full(p["encnet_W1"].shape), _full((1, EMB)),
                  _full(p["encnet_W2"].shape), _full((1, EMB)),
                  _full(L0["phi_W1"].shape), _full((1, EMB)),
                  _full(L0["phi_W2"].shape), _full((1, EMB)),
                  _full(L0["psi_W1"].shape), _full((1, EMB)),
                  _full(L0["psi_W2"].shape), _full((1, EMB)),
                  _full(L0["mlp_W1"].shape), _full((1, 3 * EMB)),
                  _full(L0["mlp_W2"].shape), _full((1, EMB)),
                  _full((1, EMB)), _full((1, EMB)),
                  _full(q0["W1"].shape), _full((1, 2 * EMB)),
                  _full(q0["W2"].shape), _full((1, EMB))],
        out_specs=[_rows(BR, EMB), _rows(BR, EMB), _rows(BR, EMB),
                   _rows(BR, N_NET), _rows(BR, N_NET),
                   _full((NUM_VN, 2 * EMB)), _full((NUM_VN, EMB))],
        out_shape=[jax.ShapeDtypeStruct((N_INST, EMB), jnp.float32),
                   jax.ShapeDtypeStruct((N_INST, EMB), jnp.float32),
                   jax.ShapeDtypeStruct((N_INST, EMB), jnp.float32),
                   jax.ShapeDtypeStruct((N_INST, N_NET), jnp.int8),
                   jax.ShapeDtypeStruct((N_INST, N_NET), jnp.int8),
                   jax.ShapeDtypeStruct((NUM_VN, 2 * EMB), jnp.float32),
                   jax.ShapeDtypeStruct((NUM_VN, EMB), jnp.float32)],
    )(x, x_net, inst_net_adj_v_drive, inst_net_adj_v_sink, oh, vn0,
      p["enc_W1"], r2(p["enc_b1"]), p["enc_W2"], r2(p["enc_b2"]),
      p["encnet_W1"], r2(p["encnet_b1"]), p["encnet_W2"], r2(p["encnet_b2"]),
      L0["phi_W1"], r2(L0["phi_b1"]), L0["phi_W2"], r2(L0["phi_b2"]),
      L0["psi_W1"], r2(L0["psi_b1"]), L0["psi_W2"], r2(L0["psi_b2"]),
      L0["mlp_W1"], r2(L0["mlp_b1"]), L0["mlp_W2"], r2(L0["mlp_b2"]),
      r2(L0["ln_g"]), r2(L0["ln_b"]),
      q0["W1"], r2(q0["b1"]), q0["W2"], r2(q0["b2"]))

    na1 = pl.pallas_call(
        _net_body,
        grid=(N_NET // BN,),
        in_specs=[_rows(BN, N_INST), _full((N_INST, EMB)),
                  _rows(BN, x_net.shape[1]),
                  _full(p["encnet_W1"].shape), _full((1, EMB)),
                  _full(p["encnet_W2"].shape), _full((1, EMB)),
                  _full(L1["phi_W1"].shape), _full((1, EMB)),
                  _full(L1["phi_W2"].shape), _full((1, EMB))],
        out_specs=_rows(BN, EMB),
        out_shape=jax.ShapeDtypeStruct((N_NET, EMB), jnp.float32),
    )(net_inst_adj, h_pre0, x_net,
      p["encnet_W1"], r2(p["encnet_b1"]), p["encnet_W2"], r2(p["encnet_b2"]),
      L1["phi_W1"], r2(L1["phi_b1"]), L1["phi_W2"], r2(L1["phi_b2"]))

    h_out1, h_in1 = pl.pallas_call(
        _layer1_body,
        grid=(N_INST // BC,),
        in_specs=[_rows(BC, N_NET), _rows(BC, N_NET), _full((N_NET, EMB)),
                  _rows(BC, EMB), _rows(BC, NUM_VN), _full((NUM_VN, EMB)),
                  _full(L1["psi_W1"].shape), _full((1, EMB)),
                  _full(L1["psi_W2"].shape), _full((1, EMB)),
                  _full(L1["mlp_W1"].shape), _full((1, 3 * EMB)),
                  _full(L1["mlp_W2"].shape), _full((1, EMB)),
                  _full((1, EMB)), _full((1, EMB))],
        out_specs=[_rows(BC, EMB), _rows(BC, EMB)],
        out_shape=[jax.ShapeDtypeStruct((N_INST, EMB), jnp.float32),
                   jax.ShapeDtypeStruct((N_INST, EMB), jnp.float32)],
    )(drive_i8, sink_i8, na1, h_out0, oh, vn1,
      L1["psi_W1"], r2(L1["psi_b1"]), L1["psi_W2"], r2(L1["psi_b2"]),
      L1["mlp_W1"], r2(L1["mlp_b1"]), L1["mlp_W2"], r2(L1["mlp_b2"]),
      r2(L1["ln_g"]), r2(L1["ln_b"]))

    return jnp.concatenate([h_in0, h_in1, h_out1], axis=1)
